# Initial kernel scaffold; baseline (speedup 1.0000x reference)
import jax
import jax.numpy as jnp
from jax.experimental import pallas as pl


def kernel(x, positions, Wqkv, bqkv, Wo, bo):
    raise NotImplementedError("write your pallas kernel here")


# speedup vs baseline: 12.6982x; 12.6982x over previous
import jax
import jax.numpy as jnp
import numpy as np
from jax.experimental import pallas as pl

_B, _L, _D, _H, _K = 2, 2048, 768, 12, 16
_DH = _D // _H
_QB = 256

_INTERPRET = False


def _qkv_kernel(x_ref, w_ref, b_ref, q_ref, k_ref, v_ref):
    x = x_ref[0]
    w = w_ref[...]
    b = b_ref[...]
    qkv = jax.lax.dot_general(
        x, w, (((1,), (1,)), ((), ())),
        preferred_element_type=jnp.float32) + b
    q_ref[0] = qkv[:, 0:_D] * np.float32(1.0 / np.sqrt(_DH))
    k_ref[0] = qkv[:, _D:2 * _D]
    v_ref[0] = qkv[:, 2 * _D:3 * _D]


def _attn_kernel(pq_ref, pk_ref, q_ref, k_ref, v_ref, wo_ref, bo_ref, o_ref):
    pq = pq_ref[0]
    pk = pk_ref[0]
    d2 = jnp.zeros((_QB, _L), dtype=jnp.float32)
    for c in range(3):
        diff = pq[:, c:c + 1] - pk[c:c + 1, :]
        d2 = d2 + diff * diff
    dist = jnp.sqrt(d2)
    iota = jax.lax.broadcasted_iota(jnp.int32, (_QB, _L), 1)
    inf = jnp.float32(np.inf)
    m = dist
    keep = jnp.zeros((_QB, _L), dtype=jnp.bool_)
    for _ in range(_K):
        t = jnp.min(m, axis=1, keepdims=True)
        j = jnp.min(jnp.where(m == t, iota, jnp.int32(_L)),
                    axis=1, keepdims=True)
        hit = iota == j
        keep = keep | hit
        m = jnp.where(hit, inf, m)
    neg = jnp.where(keep, jnp.float32(0.0), -inf)

    q = q_ref[0]
    k = k_ref[0]
    v = v_ref[0]
    ctx_parts = []
    for h in range(_H):
        sl = slice(h * _DH, (h + 1) * _DH)
        s = jax.lax.dot_general(
            q[:, sl], k[:, sl], (((1,), (1,)), ((), ())),
            preferred_element_type=jnp.float32)
        s = s + neg
        mx = jnp.max(s, axis=1, keepdims=True)
        p = jnp.exp(s - mx)
        p = p / jnp.sum(p, axis=1, keepdims=True)
        ctx_parts.append(jax.lax.dot_general(
            p, v[:, sl], (((1,), (0,)), ((), ())),
            preferred_element_type=jnp.float32))
    ctx = jnp.concatenate(ctx_parts, axis=1)
    out = jax.lax.dot_general(
        ctx, wo_ref[...], (((1,), (1,)), ((), ())),
        preferred_element_type=jnp.float32) + bo_ref[...]
    o_ref[0] = out


def kernel(x, positions, Wqkv, bqkv, Wo, bo):
    nq = _L // _QB
    posq = jnp.pad(positions, ((0, 0), (0, 0), (0, 5)))
    posk = jnp.transpose(posq, (0, 2, 1))
    bq2 = bqkv.reshape(1, 3 * _D)
    bo2 = bo.reshape(1, _D)

    q, k, v = pl.pallas_call(
        _qkv_kernel,
        grid=(_B, nq),
        in_specs=[
            pl.BlockSpec((1, _QB, _D), lambda b, i: (b, i, 0)),
            pl.BlockSpec((3 * _D, _D), lambda b, i: (0, 0)),
            pl.BlockSpec((1, 3 * _D), lambda b, i: (0, 0)),
        ],
        out_specs=[
            pl.BlockSpec((1, _QB, _D), lambda b, i: (b, i, 0)),
            pl.BlockSpec((1, _QB, _D), lambda b, i: (b, i, 0)),
            pl.BlockSpec((1, _QB, _D), lambda b, i: (b, i, 0)),
        ],
        out_shape=[jax.ShapeDtypeStruct((_B, _L, _D), jnp.float32)] * 3,
        interpret=_INTERPRET,
    )(x, Wqkv, bq2)

    out = pl.pallas_call(
        _attn_kernel,
        grid=(_B, nq),
        in_specs=[
            pl.BlockSpec((1, _QB, 8), lambda b, i: (b, i, 0)),
            pl.BlockSpec((1, 8, _L), lambda b, i: (b, 0, 0)),
            pl.BlockSpec((1, _QB, _D), lambda b, i: (b, i, 0)),
            pl.BlockSpec((1, _L, _D), lambda b, i: (b, 0, 0)),
            pl.BlockSpec((1, _L, _D), lambda b, i: (b, 0, 0)),
            pl.BlockSpec((_D, _D), lambda b, i: (0, 0)),
            pl.BlockSpec((1, _D), lambda b, i: (0, 0)),
        ],
        out_specs=pl.BlockSpec((1, _QB, _D), lambda b, i: (b, i, 0)),
        out_shape=jax.ShapeDtypeStruct((_B, _L, _D), jnp.float32),
        interpret=_INTERPRET,
    )(posq, posk, q, k, v, Wo, bo2)
    return out


# speedup vs baseline: 18.8720x; 1.4862x over previous
import jax
import jax.numpy as jnp
import numpy as np
from jax.experimental import pallas as pl

_B, _L, _D, _H, _K = 2, 2048, 768, 12, 16
_DH = _D // _H
_QB = 256

_INTERPRET = False


def _qkv_kernel(x_ref, w_ref, b_ref, q_ref, k_ref, v_ref):
    x = x_ref[0]
    w = w_ref[...]
    b = b_ref[...]
    qkv = jax.lax.dot_general(
        x, w, (((1,), (1,)), ((), ())),
        preferred_element_type=jnp.float32) + b
    q_ref[0] = qkv[:, 0:_D] * np.float32(1.0 / np.sqrt(_DH))
    k_ref[0] = qkv[:, _D:2 * _D]
    v_ref[0] = qkv[:, 2 * _D:3 * _D]


def _attn_kernel(pq_ref, pk_ref, q_ref, k_ref, v_ref, wo_ref, bo_ref, o_ref):
    pq = pq_ref[0]
    pk = pk_ref[0]
    d2 = jnp.zeros((_QB, _L), dtype=jnp.float32)
    for c in range(3):
        diff = pq[:, c:c + 1] - pk[c:c + 1, :]
        d2 = d2 + diff * diff
    dist = jnp.sqrt(d2)
    inf = jnp.float32(np.inf)

    m = dist
    t = None
    for _ in range(_K):
        t = jnp.min(m, axis=1, keepdims=True)
        m = jnp.where(m <= t, inf, m)
    count = jnp.sum(jnp.where(dist <= t, 1.0, 0.0), axis=1, keepdims=True)
    has_ties = jnp.any(count != np.float32(_K))

    def _exact_topk(_):
        iota = jax.lax.broadcasted_iota(jnp.int32, (_QB, _L), 1)
        mm = dist
        keep = jnp.zeros((_QB, _L), dtype=jnp.bool_)
        for _ in range(_K):
            tt = jnp.min(mm, axis=1, keepdims=True)
            j = jnp.min(jnp.where(mm == tt, iota, jnp.int32(_L)),
                        axis=1, keepdims=True)
            hit = iota == j
            keep = keep | hit
            mm = jnp.where(hit, inf, mm)
        return jnp.where(keep, jnp.float32(0.0), -inf)

    def _thresh_mask(_):
        return jnp.where(dist <= t, jnp.float32(0.0), -inf)

    neg = jax.lax.cond(has_ties, _exact_topk, _thresh_mask, operand=None)

    q = q_ref[0]
    k = k_ref[0]
    v = v_ref[0]
    ctx_parts = []
    for h in range(_H):
        sl = slice(h * _DH, (h + 1) * _DH)
        s = jax.lax.dot_general(
            q[:, sl], k[:, sl], (((1,), (1,)), ((), ())),
            preferred_element_type=jnp.float32)
        s = s + neg
        mx = jnp.max(s, axis=1, keepdims=True)
        p = jnp.exp(s - mx)
        denom = jnp.sum(p, axis=1, keepdims=True)
        ctx_h = jax.lax.dot_general(
            p, v[:, sl], (((1,), (0,)), ((), ())),
            preferred_element_type=jnp.float32)
        ctx_parts.append(ctx_h / denom)
    ctx = jnp.concatenate(ctx_parts, axis=1)
    out = jax.lax.dot_general(
        ctx, wo_ref[...], (((1,), (1,)), ((), ())),
        preferred_element_type=jnp.float32) + bo_ref[...]
    o_ref[0] = out


def kernel(x, positions, Wqkv, bqkv, Wo, bo):
    nq = _L // _QB
    posq = jnp.pad(positions, ((0, 0), (0, 0), (0, 5)))
    posk = jnp.transpose(posq, (0, 2, 1))
    bq2 = bqkv.reshape(1, 3 * _D)
    bo2 = bo.reshape(1, _D)

    q, k, v = pl.pallas_call(
        _qkv_kernel,
        grid=(_B, nq),
        in_specs=[
            pl.BlockSpec((1, _QB, _D), lambda b, i: (b, i, 0)),
            pl.BlockSpec((3 * _D, _D), lambda b, i: (0, 0)),
            pl.BlockSpec((1, 3 * _D), lambda b, i: (0, 0)),
        ],
        out_specs=[
            pl.BlockSpec((1, _QB, _D), lambda b, i: (b, i, 0)),
            pl.BlockSpec((1, _QB, _D), lambda b, i: (b, i, 0)),
            pl.BlockSpec((1, _QB, _D), lambda b, i: (b, i, 0)),
        ],
        out_shape=[jax.ShapeDtypeStruct((_B, _L, _D), jnp.float32)] * 3,
        interpret=_INTERPRET,
    )(x, Wqkv, bq2)

    out = pl.pallas_call(
        _attn_kernel,
        grid=(_B, nq),
        in_specs=[
            pl.BlockSpec((1, _QB, 8), lambda b, i: (b, i, 0)),
            pl.BlockSpec((1, 8, _L), lambda b, i: (b, 0, 0)),
            pl.BlockSpec((1, _QB, _D), lambda b, i: (b, i, 0)),
            pl.BlockSpec((1, _L, _D), lambda b, i: (b, 0, 0)),
            pl.BlockSpec((1, _L, _D), lambda b, i: (b, 0, 0)),
            pl.BlockSpec((_D, _D), lambda b, i: (0, 0)),
            pl.BlockSpec((1, _D), lambda b, i: (0, 0)),
        ],
        out_specs=pl.BlockSpec((1, _QB, _D), lambda b, i: (b, i, 0)),
        out_shape=jax.ShapeDtypeStruct((_B, _L, _D), jnp.float32),
        interpret=_INTERPRET,
    )(posq, posk, q, k, v, Wo, bo2)
    return out


# speedup vs baseline: 21.4618x; 1.1372x over previous
import jax
import jax.numpy as jnp
import numpy as np
from jax.experimental import pallas as pl

_B, _L, _D, _H, _K = 2, 2048, 768, 12, 16
_DH = _D // _H
_QB = 256

_INTERPRET = False


def _qkv_kernel(x_ref, w_ref, b_ref, q_ref, k_ref, v_ref):
    x = x_ref[0]
    w = w_ref[...]
    b = b_ref[...]
    qkv = jax.lax.dot_general(
        x, w, (((1,), (1,)), ((), ())),
        preferred_element_type=jnp.float32) + b
    q_ref[0] = qkv[:, 0:_D] * np.float32(1.0 / np.sqrt(_DH))
    k_ref[0] = qkv[:, _D:2 * _D]
    v_ref[0] = qkv[:, 2 * _D:3 * _D]


def _attn_kernel(pq_ref, pk_ref, q_ref, k_ref, v_ref, wo_ref, bo_ref, o_ref):
    pq = pq_ref[0]
    pk = pk_ref[0]
    d2 = jnp.zeros((_QB, _L), dtype=jnp.float32)
    for c in range(3):
        diff = pq[:, c:c + 1] - pk[c:c + 1, :]
        d2 = d2 + diff * diff
    dist = jnp.sqrt(d2)
    inf = jnp.float32(np.inf)

    a1 = jnp.full((_QB, 128), inf, dtype=jnp.float32)
    a2 = jnp.full((_QB, 128), inf, dtype=jnp.float32)
    a3 = jnp.full((_QB, 128), inf, dtype=jnp.float32)
    a4 = jnp.full((_QB, 128), inf, dtype=jnp.float32)
    for g in range(_L // 128):
        sg = dist[:, g * 128:(g + 1) * 128]
        t2 = jnp.maximum(a1, sg)
        a1 = jnp.minimum(a1, sg)
        t3 = jnp.maximum(a2, t2)
        a2 = jnp.minimum(a2, t2)
        t4 = jnp.maximum(a3, t3)
        a3 = jnp.minimum(a3, t3)
        a4 = jnp.minimum(a4, t4)
    m = jnp.concatenate([a1, a2, a3, a4], axis=1)
    t = None
    for _ in range(_K):
        t = jnp.min(m, axis=1, keepdims=True)
        m = jnp.where(m <= t, inf, m)
    count = jnp.sum(jnp.where(dist <= t, 1.0, 0.0), axis=1, keepdims=True)
    has_ties = jnp.any(count != np.float32(_K))

    def _exact_topk(_):
        iota = jax.lax.broadcasted_iota(jnp.int32, (_QB, _L), 1)
        mm = dist
        keep = jnp.zeros((_QB, _L), dtype=jnp.bool_)
        for _ in range(_K):
            tt = jnp.min(mm, axis=1, keepdims=True)
            j = jnp.min(jnp.where(mm == tt, iota, jnp.int32(_L)),
                        axis=1, keepdims=True)
            hit = iota == j
            keep = keep | hit
            mm = jnp.where(hit, inf, mm)
        return jnp.where(keep, jnp.float32(0.0), -inf)

    def _thresh_mask(_):
        return jnp.where(dist <= t, jnp.float32(0.0), -inf)

    neg = jax.lax.cond(has_ties, _exact_topk, _thresh_mask, operand=None)

    q = q_ref[0]
    k = k_ref[0]
    v = v_ref[0]
    ones_col = jnp.ones((_L, 1), dtype=jnp.float32)
    ctx_parts = []
    for h in range(_H):
        sl = slice(h * _DH, (h + 1) * _DH)
        s = jax.lax.dot_general(
            q[:, sl], k[:, sl], (((1,), (1,)), ((), ())),
            preferred_element_type=jnp.float32)
        s = s + neg
        mx = jnp.max(s, axis=1, keepdims=True)
        p = jnp.exp(s - mx)
        v_aug = jnp.concatenate([v[:, sl], ones_col], axis=1)
        ctx_aug = jax.lax.dot_general(
            p, v_aug, (((1,), (0,)), ((), ())),
            preferred_element_type=jnp.float32)
        ctx_parts.append(ctx_aug[:, :_DH] / ctx_aug[:, _DH:_DH + 1])
    ctx = jnp.concatenate(ctx_parts, axis=1)
    out = jax.lax.dot_general(
        ctx, wo_ref[...], (((1,), (1,)), ((), ())),
        preferred_element_type=jnp.float32) + bo_ref[...]
    o_ref[0] = out


def kernel(x, positions, Wqkv, bqkv, Wo, bo):
    nq = _L // _QB
    posq = jnp.pad(positions, ((0, 0), (0, 0), (0, 5)))
    posk = jnp.transpose(posq, (0, 2, 1))
    bq2 = bqkv.reshape(1, 3 * _D)
    bo2 = bo.reshape(1, _D)

    q, k, v = pl.pallas_call(
        _qkv_kernel,
        grid=(_B, nq),
        in_specs=[
            pl.BlockSpec((1, _QB, _D), lambda b, i: (b, i, 0)),
            pl.BlockSpec((3 * _D, _D), lambda b, i: (0, 0)),
            pl.BlockSpec((1, 3 * _D), lambda b, i: (0, 0)),
        ],
        out_specs=[
            pl.BlockSpec((1, _QB, _D), lambda b, i: (b, i, 0)),
            pl.BlockSpec((1, _QB, _D), lambda b, i: (b, i, 0)),
            pl.BlockSpec((1, _QB, _D), lambda b, i: (b, i, 0)),
        ],
        out_shape=[jax.ShapeDtypeStruct((_B, _L, _D), jnp.float32)] * 3,
        interpret=_INTERPRET,
    )(x, Wqkv, bq2)

    out = pl.pallas_call(
        _attn_kernel,
        grid=(_B, nq),
        in_specs=[
            pl.BlockSpec((1, _QB, 8), lambda b, i: (b, i, 0)),
            pl.BlockSpec((1, 8, _L), lambda b, i: (b, 0, 0)),
            pl.BlockSpec((1, _QB, _D), lambda b, i: (b, i, 0)),
            pl.BlockSpec((1, _L, _D), lambda b, i: (b, 0, 0)),
            pl.BlockSpec((1, _L, _D), lambda b, i: (b, 0, 0)),
            pl.BlockSpec((_D, _D), lambda b, i: (0, 0)),
            pl.BlockSpec((1, _D), lambda b, i: (0, 0)),
        ],
        out_specs=pl.BlockSpec((1, _QB, _D), lambda b, i: (b, i, 0)),
        out_shape=jax.ShapeDtypeStruct((_B, _L, _D), jnp.float32),
        interpret=_INTERPRET,
    )(posq, posk, q, k, v, Wo, bo2)
    return out
